# baseline (speedup 1.0000x reference)
import jax
import jax.numpy as jnp
from jax import lax
from jax.experimental import pallas as pl
from jax.experimental.pallas import tpu as pltpu
from jax.experimental.pallas import tpu_sc as plsc

N = 10000
E = 160000
D_IN = 128
HEADS = 8
HID = 64
DM = HEADS * HID
NHP = 4
NT = 16
ET = E // NT
C = 80
NCH = ET // C
CA = 48
CB = C - CA
NACC = 10240
NPT = NACC // NT
ND = NACC // 64
BN1 = 400
NB1 = N // BN1
BN2 = 512
NB2 = NACC // BN2


def _f16x2_to_f32(v):
    def cvt(h):
        sign = lax.shift_left(jnp.bitwise_and(h, 0x8000), 16)
        mag = lax.shift_left(jnp.bitwise_and(h, 0x7FFF), 13) + (112 << 23)
        mag = jnp.where(jnp.bitwise_and(h, 0x7FFF) == 0, 0, mag)
        return lax.bitcast_convert_type(jnp.bitwise_or(sign, mag),
                                        jnp.float32)
    h0 = jnp.bitwise_and(v, 0xFFFF)
    h1 = jnp.bitwise_and(lax.shift_right_logical(v, 16), 0xFFFF)
    return cvt(h0), cvt(h1)


def _sc_edge_body(h4, ab4, eidx, zrows, out_num,
                  table_v, eidx_v, didx, didx2a, didx2b, gbufa, gbufb,
                  dstage, accum, sema, semb, semfa, semfb, semd, semi):
    core = lax.axis_index("c")
    s = lax.axis_index("s")

    def _dzrow(i, _):
        for v in range(128 // 16):
            dstage[i, pl.ds(v * 16, 16)] = jnp.zeros((16,), jnp.float32)
        return 0
    lax.fori_loop(0, C, _dzrow, 0)

    for p in range(2):
        hp = core * 2 + p
        pltpu.sync_copy(ab4.at[hp], table_v)
        for k in range(NPT // 128):
            pltpu.sync_copy(zrows, accum.at[pl.ds(s * NPT + k * 128, 128)])
        pltpu.sync_copy(eidx.at[pl.ds((s * NCH) * 2 * C, 2 * C)],
                        eidx_v.at[pl.ds(0, 2 * C)])
        plsc.subcore_barrier()

        def _rezero(d2ref, k0, k1, base):
            zf = jnp.zeros((16,), jnp.float32)
            for k in range(k0, k1):
                dst16 = d2ref[pl.ds(k * 16 - base, 16)]
                c0 = lax.shift_left(jnp.bitwise_and(dst16, 63), 1)
                rows = lax.iota(jnp.int32, 16) + (k * 16)
                plsc.store_scatter(dstage, [rows, c0], zf)
                plsc.store_scatter(dstage, [rows, c0 + 1], zf)

        def _chunk(j, _, hp=hp):
            b = jnp.bitwise_and(j, 1) * (2 * C)
            nb = (2 * C) - b

            @pl.when(j > 0)
            def _():
                pltpu.make_async_copy(
                    eidx.at[pl.ds((s * NCH + j) * 2 * C, 2 * C)],
                    eidx_v.at[pl.ds(b, 2 * C)], semi).wait()

            @pl.when(j + 1 < NCH)
            def _():
                pltpu.async_copy(
                    eidx.at[pl.ds((s * NCH + j + 1) * 2 * C, 2 * C)],
                    eidx_v.at[pl.ds(nb, 2 * C)], semi)
            @pl.when(j > 0)
            def _():
                pltpu.make_async_copy(gbufa, accum.at[didx2a],
                                      semfa).wait()
            ga = pltpu.async_copy(h4.at[hp].at[eidx_v.at[pl.ds(b, CA)]],
                                  gbufa, sema)

            @pl.when(j > 0)
            def _():
                pltpu.make_async_copy(gbufb, accum.at[didx2b],
                                      semfb).wait()
            gb = pltpu.async_copy(h4.at[hp].at[eidx_v.at[pl.ds(b + CA, CB)]],
                                  gbufb, semb)

            @pl.when(j > 0)
            def _():
                pltpu.make_async_copy(dstage, accum.at[didx], semd).wait()
                _rezero(didx2a, 0, CA // 16, 0)
                _rezero(didx2b, CA // 16, C // 16, CA)

            def _groups(gref, d2ref, k0, k1, base):
                for k in range(k0, k1):
                    src16 = eidx_v[pl.ds(b + k * 16, 16)]
                    dst16 = eidx_v[pl.ds(b + C + k * 16, 16)]
                    sg = plsc.load_gather(table_v, [src16 * 2])
                    dg = plsc.load_gather(table_v, [dst16 * 2 + 1])
                    sa0, sa1 = _f16x2_to_f32(sg)
                    da0, da1 = _f16x2_to_f32(dg)
                    e0 = sa0 + da0
                    e1 = sa1 + da1
                    w0 = jnp.exp(jnp.where(e0 > 0, e0, e0 * 0.2))
                    w1 = jnp.exp(jnp.where(e1 > 0, e1, e1 * 0.2))
                    rows = lax.iota(jnp.int32, 16) + (k * 16)
                    didx[pl.ds(k * 16, 16)] = (
                        lax.shift_right_logical(dst16, 6) + N)
                    d2ref[pl.ds(k * 16 - base, 16)] = dst16
                    c0 = lax.shift_left(jnp.bitwise_and(dst16, 63), 1)
                    plsc.store_scatter(dstage, [rows, c0], w0)
                    plsc.store_scatter(dstage, [rows, c0 + 1], w1)
                    for el in range(16):
                        e = k * 16 + el - base
                        w0e = w0[el]
                        w1e = w1[el]
                        for v in range(8):
                            we = w0e if v < 4 else w1e
                            gref[e, pl.ds(v * 16, 16)] = (
                                gref[e, pl.ds(v * 16, 16)] * we)

            ga.wait()
            _groups(gbufa, didx2a, 0, CA // 16, 0)
            pltpu.async_copy(gbufa, accum.at[didx2a], semfa, add=True)
            gb.wait()
            _groups(gbufb, didx2b, CA // 16, C // 16, CA)
            pltpu.async_copy(gbufb, accum.at[didx2b], semfb, add=True)
            pltpu.async_copy(dstage, accum.at[didx], semd, add=True)
            return 0

        lax.fori_loop(0, NCH, _chunk, 0)
        pltpu.make_async_copy(gbufa, accum.at[didx2a], semfa).wait()
        pltpu.make_async_copy(gbufb, accum.at[didx2b], semfb).wait()
        pltpu.make_async_copy(dstage, accum.at[didx], semd).wait()
        _rezero(didx2a, 0, CA // 16, 0)
        _rezero(didx2b, CA // 16, C // 16, CA)
        plsc.subcore_barrier()
        pltpu.sync_copy(accum.at[pl.ds(s * NPT, NPT)],
                        out_num.at[hp, pl.ds(s * NPT, NPT)])
        plsc.subcore_barrier()


def _sc_edge_layer(h4, ab4, eidx, zrows):
    return pl.kernel(
        _sc_edge_body,
        out_type=jax.ShapeDtypeStruct((NHP, NACC, 128), jnp.float32),
        mesh=plsc.VectorSubcoreMesh(core_axis_name="c", subcore_axis_name="s",
                                    num_cores=2, num_subcores=16),
        compiler_params=pltpu.CompilerParams(needs_layout_passes=False),
        scratch_types=[
            pltpu.VMEM((N * 2,), jnp.int32),
            pltpu.VMEM((4 * C,), jnp.int32),
            pltpu.VMEM((C,), jnp.int32),
            pltpu.VMEM((CA,), jnp.int32),
            pltpu.VMEM((CB,), jnp.int32),
            pltpu.VMEM((CA, 128), jnp.float32),
            pltpu.VMEM((CB, 128), jnp.float32),
            pltpu.VMEM((C, 128), jnp.float32),
            pltpu.VMEM_SHARED((NACC, 128), jnp.float32),
            pltpu.SemaphoreType.DMA,
            pltpu.SemaphoreType.DMA,
            pltpu.SemaphoreType.DMA,
            pltpu.SemaphoreType.DMA,
            pltpu.SemaphoreType.DMA,
            pltpu.SemaphoreType.DMA,
        ],
    )(h4, ab4, eidx, zrows)


def _tc_in_body(x_ref, w_ref, abm_ref, h_ref, ab_ref):
    j = pl.program_id(1)
    hb = jnp.dot(x_ref[...], w_ref[...], preferred_element_type=jnp.float32)
    h_ref[0] = hb
    contrib = jnp.dot(hb, abm_ref[...], preferred_element_type=jnp.float32)

    @pl.when(j == 0)
    def _():
        ab_ref[...] = contrib

    @pl.when(j > 0)
    def _():
        ab_ref[...] += contrib


def _tc_in(x, W, ABm):
    din = x.shape[1]
    return pl.pallas_call(
        _tc_in_body,
        grid=(NB1, NHP),
        in_specs=[
            pl.BlockSpec((BN1, din), lambda i, j: (i, 0)),
            pl.BlockSpec((din, 128), lambda i, j: (0, j)),
            pl.BlockSpec((128, 16), lambda i, j: (j, 0)),
        ],
        out_specs=[
            pl.BlockSpec((1, BN1, 128), lambda i, j: (j, i, 0)),
            pl.BlockSpec((BN1, 16), lambda i, j: (i, 0)),
        ],
        out_shape=[
            jax.ShapeDtypeStruct((NHP, N, 128), jnp.float32),
            jax.ShapeDtypeStruct((N, 16), jnp.float32),
        ],
    )(x, W, ABm)


def _tc_t_block(num_ref, den_ref, b_ref):
    parts = []
    for hp in range(NHP):
        nb = num_ref[hp]
        d0 = den_ref[hp][:, 0:1] + 1e-16
        d1 = den_ref[hp][:, 1:2] + 1e-16
        parts.append(nb[:, 0:64] / d0)
        parts.append(nb[:, 64:128] / d1)
    return jnp.concatenate(parts, axis=1) + b_ref[...]


def _tc_sums(i, t, su_sc, ss_sc):
    @pl.when(i == 0)
    def _():
        su_sc[...] = jnp.zeros_like(su_sc)
        ss_sc[...] = jnp.zeros_like(ss_sc)

    row = lax.broadcasted_iota(jnp.int32, (BN2, DM), 0) + i * BN2
    tm = jnp.where(row < N, t, 0.0)
    su_sc[...] += jnp.sum(tm, axis=0, keepdims=True)
    ss_sc[...] += jnp.sum(tm * tm, axis=0, keepdims=True)


def _tc_bn(t, su_sc, ss_sc, g_ref, be_ref):
    mean = su_sc[...] / N
    var = ss_sc[...] / N - mean * mean
    return g_ref[...] * (t - mean) * lax.rsqrt(var + 1e-5) + be_ref[...]


def _tc_nm_body(num_ref, den_ref, b_ref, g_ref, be_ref, w_ref, abm_ref,
                h_ref, ab_ref, su_sc, ss_sc):
    ph = pl.program_id(0)
    i = pl.program_id(1)
    t = _tc_t_block(num_ref, den_ref, b_ref)

    @pl.when(ph == 0)
    def _():
        _tc_sums(i, t, su_sc, ss_sc)

    @pl.when(ph == 1)
    def _():
        xh = _tc_bn(t, su_sc, ss_sc, g_ref, be_ref)
        u = jnp.where(xh > 0, xh, jnp.exp(xh) - 1.0)
        h1 = jnp.dot(u, w_ref[...], preferred_element_type=jnp.float32)
        for hp in range(NHP):
            h_ref[hp] = h1[:, hp * 128:(hp + 1) * 128]
        ab_ref[...] = jnp.dot(h1, abm_ref[...],
                              preferred_element_type=jnp.float32)


def _tc_nm(num, den, b, g, be, W, ABm):
    return pl.pallas_call(
        _tc_nm_body,
        grid=(2, NB2),
        in_specs=[
            pl.BlockSpec((NHP, BN2, 128), lambda ph, i: (0, i, 0)),
            pl.BlockSpec((NHP, BN2, 2), lambda ph, i: (0, i, 0)),
            pl.BlockSpec((1, DM), lambda ph, i: (0, 0)),
            pl.BlockSpec((1, DM), lambda ph, i: (0, 0)),
            pl.BlockSpec((1, DM), lambda ph, i: (0, 0)),
            pl.BlockSpec((DM, DM), lambda ph, i: (0, 0)),
            pl.BlockSpec((DM, 16), lambda ph, i: (0, 0)),
        ],
        out_specs=[
            pl.BlockSpec((NHP, BN2, 128), lambda ph, i: (0, i * ph, 0)),
            pl.BlockSpec((BN2, 16), lambda ph, i: (i * ph, 0)),
        ],
        out_shape=[
            jax.ShapeDtypeStruct((NHP, NACC, 128), jnp.float32),
            jax.ShapeDtypeStruct((NACC, 16), jnp.float32),
        ],
        scratch_shapes=[
            pltpu.VMEM((1, DM), jnp.float32),
            pltpu.VMEM((1, DM), jnp.float32),
        ],
    )(num, den, b, g, be, W, ABm)


def _tc_no_body(num_ref, den_ref, b_ref, g_ref, be_ref, w_ref, bc_ref,
                o_ref, su_sc, ss_sc):
    ph = pl.program_id(0)
    i = pl.program_id(1)
    t = _tc_t_block(num_ref, den_ref, b_ref)

    @pl.when(ph == 0)
    def _():
        _tc_sums(i, t, su_sc, ss_sc)

    @pl.when(ph == 1)
    def _():
        xh = _tc_bn(t, su_sc, ss_sc, g_ref, be_ref)
        o_ref[...] = jnp.dot(xh, w_ref[...],
                             preferred_element_type=jnp.float32) + bc_ref[...]


def _tc_no(num, den, b, g, be, Wp, bp):
    return pl.pallas_call(
        _tc_no_body,
        grid=(2, NB2),
        in_specs=[
            pl.BlockSpec((NHP, BN2, 128), lambda ph, i: (0, i, 0)),
            pl.BlockSpec((NHP, BN2, 2), lambda ph, i: (0, i, 0)),
            pl.BlockSpec((1, DM), lambda ph, i: (0, 0)),
            pl.BlockSpec((1, DM), lambda ph, i: (0, 0)),
            pl.BlockSpec((1, DM), lambda ph, i: (0, 0)),
            pl.BlockSpec((DM, 128), lambda ph, i: (0, 0)),
            pl.BlockSpec((1, 128), lambda ph, i: (0, 0)),
        ],
        out_specs=pl.BlockSpec((BN2, 128), lambda ph, i: (i * ph, 0)),
        out_shape=jax.ShapeDtypeStruct((NACC, 128), jnp.float32),
        scratch_shapes=[
            pltpu.VMEM((1, DM), jnp.float32),
            pltpu.VMEM((1, DM), jnp.float32),
        ],
    )(num, den, b, g, be, Wp, bp)


def _mixing_matrix(a_src, a_dst):
    idx = jnp.arange(HEADS)
    asr = jnp.zeros((HEADS, HID, HEADS), jnp.float32).at[idx, :, idx].set(a_src)
    ads = jnp.zeros((HEADS, HID, HEADS), jnp.float32).at[idx, :, idx].set(a_dst)
    asr = asr.reshape(DM, HEADS)
    ads = ads.reshape(DM, HEADS)
    cols = []
    for hp in range(NHP):
        cols += [asr[:, 2 * hp], asr[:, 2 * hp + 1],
                 ads[:, 2 * hp], ads[:, 2 * hp + 1]]
    return jnp.stack(cols, axis=1)


def _pack_alpha(ab):
    a = ab.reshape(ab.shape[0], NHP, 2, 2).astype(jnp.float16)
    packed = jax.lax.bitcast_convert_type(a, jnp.int32)
    return packed.transpose(1, 0, 2).reshape(NHP, ab.shape[0] * 2)


def kernel(x, edge_index, W0, a_src0, a_dst0, b0, g0, be0,
           W1, a_src1, a_dst1, b1, g1, be1, Wc, bc):
    eidx = (jnp.stack([edge_index[0].reshape(NT, NCH, C),
                       edge_index[1].reshape(NT, NCH, C)], axis=2)
            .reshape(2 * E))

    AB0 = _mixing_matrix(a_src0, a_dst0)
    AB1 = _mixing_matrix(a_src1, a_dst1)
    b0r = b0.reshape(1, DM)
    b1r = b1.reshape(1, DM)
    g0r, be0r = g0.reshape(1, DM), be0.reshape(1, DM)
    g1r, be1r = g1.reshape(1, DM), be1.reshape(1, DM)
    Wcp = jnp.zeros((DM, 128), jnp.float32).at[:, :Wc.shape[1]].set(Wc)
    bcp = jnp.zeros((1, 128), jnp.float32).at[0, :Wc.shape[1]].set(bc)

    zrows = jnp.zeros((128, 128), jnp.float32)

    h0p, ab0 = _tc_in(x, W0, AB0)
    num0 = _sc_edge_layer(h0p, _pack_alpha(ab0), eidx, zrows)
    den0 = num0[:, N:N + ND].reshape(NHP, ND * 64, 2)[:, :NACC]

    h1p, ab1 = _tc_nm(num0, den0, b0r, g0r, be0r, W1, AB1)
    num1 = _sc_edge_layer(h1p, _pack_alpha(ab1[:N]), eidx, zrows)
    den1 = num1[:, N:N + ND].reshape(NHP, ND * 64, 2)[:, :NACC]

    logits = _tc_no(num1, den1, b1r, g1r, be1r, Wcp, bcp)
    return logits[:N, :Wc.shape[1]]


# speedup vs baseline: 1.0064x; 1.0064x over previous
import jax
import jax.numpy as jnp
from jax import lax
from jax.experimental import pallas as pl
from jax.experimental.pallas import tpu as pltpu
from jax.experimental.pallas import tpu_sc as plsc

N = 10000
E = 160000
D_IN = 128
HEADS = 8
HID = 64
DM = HEADS * HID
NHP = 4
NT = 16
ET = E // NT
C = 80
NCH = ET // C
CA = 48
CB = C - CA
NACC = 10240
NPT = NACC // NT
ND = NACC // 64
BN1 = 400
NB1 = N // BN1
BN2 = 512
NB2 = NACC // BN2


def _f16x2_to_f32(v):
    def cvt(h):
        sign = lax.shift_left(jnp.bitwise_and(h, 0x8000), 16)
        mag = lax.shift_left(jnp.bitwise_and(h, 0x7FFF), 13) + (112 << 23)
        mag = jnp.where(jnp.bitwise_and(h, 0x7FFF) == 0, 0, mag)
        return lax.bitcast_convert_type(jnp.bitwise_or(sign, mag),
                                        jnp.float32)
    h0 = jnp.bitwise_and(v, 0xFFFF)
    h1 = jnp.bitwise_and(lax.shift_right_logical(v, 16), 0xFFFF)
    return cvt(h0), cvt(h1)


def _sc_edge_body(h4, ab4, eidx, zrows, out_num,
                  table_v, eidx_v, didx, didx2a, didx2b, gbufa, gbufb,
                  dstage, accum, sema, semb, semfa, semfb, semd, semi):
    core = lax.axis_index("c")
    s = lax.axis_index("s")

    def _dzrow(i, _):
        for v in range(128 // 16):
            dstage[i, pl.ds(v * 16, 16)] = jnp.zeros((16,), jnp.float32)
        return 0
    lax.fori_loop(0, C, _dzrow, 0)

    for p in range(2):
        hp = core * 2 + p
        pltpu.sync_copy(ab4.at[hp], table_v)
        for k in range(NPT // 128):
            pltpu.sync_copy(zrows, accum.at[pl.ds(s * NPT + k * 128, 128)])
        pltpu.sync_copy(eidx.at[pl.ds((s * NCH) * 2 * C, 2 * C)],
                        eidx_v.at[pl.ds(0, 2 * C)])
        plsc.subcore_barrier()

        def _rezero(d2ref, k0, k1, base):
            zf = jnp.zeros((16,), jnp.float32)
            for k in range(k0, k1):
                dst16 = d2ref[pl.ds(k * 16 - base, 16)]
                c0 = lax.shift_left(jnp.bitwise_and(dst16, 63), 1)
                rows = lax.iota(jnp.int32, 16) + (k * 16)
                plsc.store_scatter(dstage, [rows, c0], zf)
                plsc.store_scatter(dstage, [rows, c0 + 1], zf)

        def _chunk(j, _, hp=hp):
            b = jnp.bitwise_and(j, 1) * (2 * C)
            nb = (2 * C) - b

            @pl.when(j > 0)
            def _():
                pltpu.make_async_copy(
                    eidx.at[pl.ds((s * NCH + j) * 2 * C, 2 * C)],
                    eidx_v.at[pl.ds(b, 2 * C)], semi).wait()

            @pl.when(j + 1 < NCH)
            def _():
                pltpu.async_copy(
                    eidx.at[pl.ds((s * NCH + j + 1) * 2 * C, 2 * C)],
                    eidx_v.at[pl.ds(nb, 2 * C)], semi)
            @pl.when(j > 0)
            def _():
                pltpu.make_async_copy(gbufa, accum.at[didx2a],
                                      semfa).wait()
            ga = pltpu.async_copy(h4.at[hp].at[eidx_v.at[pl.ds(b, CA)]],
                                  gbufa, sema)

            @pl.when(j > 0)
            def _():
                pltpu.make_async_copy(gbufb, accum.at[didx2b],
                                      semfb).wait()
            gb = pltpu.async_copy(h4.at[hp].at[eidx_v.at[pl.ds(b + CA, CB)]],
                                  gbufb, semb)

            @pl.when(j > 0)
            def _():
                pltpu.make_async_copy(dstage, accum.at[didx], semd).wait()
                _rezero(didx2a, 0, CA // 16, 0)
                _rezero(didx2b, CA // 16, C // 16, CA)

            def _groups(gref, d2ref, k0, k1, base):
                for k in range(k0, k1):
                    src16 = eidx_v[pl.ds(b + k * 16, 16)]
                    dst16 = eidx_v[pl.ds(b + C + k * 16, 16)]
                    sg = plsc.load_gather(table_v, [src16 * 2])
                    dg = plsc.load_gather(table_v, [dst16 * 2 + 1])
                    sa0, sa1 = _f16x2_to_f32(sg)
                    da0, da1 = _f16x2_to_f32(dg)
                    e0 = sa0 + da0
                    e1 = sa1 + da1
                    w0 = jnp.exp(jnp.where(e0 > 0, e0, e0 * 0.2))
                    w1 = jnp.exp(jnp.where(e1 > 0, e1, e1 * 0.2))
                    rows = lax.iota(jnp.int32, 16) + (k * 16)
                    didx[pl.ds(k * 16, 16)] = (
                        lax.shift_right_logical(dst16, 6) + N)
                    d2ref[pl.ds(k * 16 - base, 16)] = dst16
                    c0 = lax.shift_left(jnp.bitwise_and(dst16, 63), 1)
                    plsc.store_scatter(dstage, [rows, c0], w0)
                    plsc.store_scatter(dstage, [rows, c0 + 1], w1)
                    for el in range(16):
                        e = k * 16 + el - base
                        w0e = w0[el]
                        w1e = w1[el]
                        for v in range(8):
                            we = w0e if v < 4 else w1e
                            gref[e, pl.ds(v * 16, 16)] = (
                                gref[e, pl.ds(v * 16, 16)] * we)

            ga.wait()
            _groups(gbufa, didx2a, 0, CA // 16, 0)
            pltpu.async_copy(gbufa, accum.at[didx2a], semfa, add=True)
            gb.wait()
            _groups(gbufb, didx2b, CA // 16, C // 16, CA)
            pltpu.async_copy(gbufb, accum.at[didx2b], semfb, add=True)
            pltpu.async_copy(dstage, accum.at[didx], semd, add=True)
            return 0

        lax.fori_loop(0, NCH, _chunk, 0)
        pltpu.make_async_copy(gbufa, accum.at[didx2a], semfa).wait()
        pltpu.make_async_copy(gbufb, accum.at[didx2b], semfb).wait()
        pltpu.make_async_copy(dstage, accum.at[didx], semd).wait()
        _rezero(didx2a, 0, CA // 16, 0)
        _rezero(didx2b, CA // 16, C // 16, CA)
        plsc.subcore_barrier()
        pltpu.sync_copy(accum.at[pl.ds(s * NPT, NPT)],
                        out_num.at[hp, pl.ds(s * NPT, NPT)])
        plsc.subcore_barrier()


def _sc_edge_layer(h4, ab4, eidx, zrows):
    return pl.kernel(
        _sc_edge_body,
        out_type=jax.ShapeDtypeStruct((NHP, NACC, 128), jnp.float32),
        mesh=plsc.VectorSubcoreMesh(core_axis_name="c", subcore_axis_name="s",
                                    num_cores=2, num_subcores=16),
        compiler_params=pltpu.CompilerParams(needs_layout_passes=False),
        scratch_types=[
            pltpu.VMEM((N * 2,), jnp.int32),
            pltpu.VMEM((4 * C,), jnp.int32),
            pltpu.VMEM((C,), jnp.int32),
            pltpu.VMEM((CA,), jnp.int32),
            pltpu.VMEM((CB,), jnp.int32),
            pltpu.VMEM((CA, 128), jnp.float32),
            pltpu.VMEM((CB, 128), jnp.float32),
            pltpu.VMEM((C, 128), jnp.float32),
            pltpu.VMEM_SHARED((NACC, 128), jnp.float32),
            pltpu.SemaphoreType.DMA,
            pltpu.SemaphoreType.DMA,
            pltpu.SemaphoreType.DMA,
            pltpu.SemaphoreType.DMA,
            pltpu.SemaphoreType.DMA,
            pltpu.SemaphoreType.DMA,
        ],
    )(h4, ab4, eidx, zrows)


def _tc_in_body(x_ref, w_ref, abm_ref, h_ref, ab_ref):
    j = pl.program_id(1)
    hb = jnp.dot(x_ref[...], w_ref[...], preferred_element_type=jnp.float32)
    h_ref[0] = hb
    contrib = jnp.dot(hb, abm_ref[...], preferred_element_type=jnp.float32)

    @pl.when(j == 0)
    def _():
        ab_ref[...] = contrib

    @pl.when(j > 0)
    def _():
        ab_ref[...] += contrib


def _tc_in(x, W, ABm):
    din = x.shape[1]
    return pl.pallas_call(
        _tc_in_body,
        grid=(NB1, NHP),
        in_specs=[
            pl.BlockSpec((BN1, din), lambda i, j: (i, 0)),
            pl.BlockSpec((din, 128), lambda i, j: (0, j)),
            pl.BlockSpec((128, 16), lambda i, j: (j, 0)),
        ],
        out_specs=[
            pl.BlockSpec((1, BN1, 128), lambda i, j: (j, i, 0)),
            pl.BlockSpec((BN1, 16), lambda i, j: (i, 0)),
        ],
        out_shape=[
            jax.ShapeDtypeStruct((NHP, N, 128), jnp.float32),
            jax.ShapeDtypeStruct((N, 16), jnp.float32),
        ],
    )(x, W, ABm)


def _tc_norm_body(num_ref, den_ref, b_ref, t_ref, su_ref, ss_ref):
    i = pl.program_id(0)
    parts = []
    for hp in range(NHP):
        nb = num_ref[hp]
        d0 = den_ref[hp][:, 0:1] + 1e-16
        d1 = den_ref[hp][:, 1:2] + 1e-16
        parts.append(nb[:, 0:64] / d0)
        parts.append(nb[:, 64:128] / d1)
    t = jnp.concatenate(parts, axis=1) + b_ref[...]
    t_ref[...] = t

    @pl.when(i == 0)
    def _():
        su_ref[...] = jnp.zeros_like(su_ref)
        ss_ref[...] = jnp.zeros_like(ss_ref)

    row = lax.broadcasted_iota(jnp.int32, (BN2, DM), 0) + i * BN2
    tm = jnp.where(row < N, t, 0.0)
    su_ref[...] += jnp.sum(tm, axis=0, keepdims=True)
    ss_ref[...] += jnp.sum(tm * tm, axis=0, keepdims=True)


def _tc_norm(num, den, b):
    return pl.pallas_call(
        _tc_norm_body,
        grid=(NB2,),
        in_specs=[
            pl.BlockSpec((NHP, BN2, 128), lambda i: (0, i, 0)),
            pl.BlockSpec((NHP, BN2, 2), lambda i: (0, i, 0)),
            pl.BlockSpec((1, DM), lambda i: (0, 0)),
        ],
        out_specs=[
            pl.BlockSpec((BN2, DM), lambda i: (i, 0)),
            pl.BlockSpec((1, DM), lambda i: (0, 0)),
            pl.BlockSpec((1, DM), lambda i: (0, 0)),
        ],
        out_shape=[
            jax.ShapeDtypeStruct((NACC, DM), jnp.float32),
            jax.ShapeDtypeStruct((1, DM), jnp.float32),
            jax.ShapeDtypeStruct((1, DM), jnp.float32),
        ],
    )(num, den, b)


def _tc_mid_body(t_ref, su_ref, ss_ref, g_ref, be_ref, w_ref, abm_ref,
                 h_ref, ab_ref):
    mean = su_ref[...] / N
    var = ss_ref[...] / N - mean * mean
    xh = g_ref[...] * (t_ref[...] - mean) * lax.rsqrt(var + 1e-5) + be_ref[...]
    u = jnp.where(xh > 0, xh, jnp.exp(xh) - 1.0)
    h1 = jnp.dot(u, w_ref[...], preferred_element_type=jnp.float32)
    for hp in range(NHP):
        h_ref[hp] = h1[:, hp * 128:(hp + 1) * 128]
    ab_ref[...] = jnp.dot(h1, abm_ref[...], preferred_element_type=jnp.float32)


def _tc_mid(t, su, ss, g, be, W, ABm):
    return pl.pallas_call(
        _tc_mid_body,
        grid=(NB2,),
        in_specs=[
            pl.BlockSpec((BN2, DM), lambda i: (i, 0)),
            pl.BlockSpec((1, DM), lambda i: (0, 0)),
            pl.BlockSpec((1, DM), lambda i: (0, 0)),
            pl.BlockSpec((1, DM), lambda i: (0, 0)),
            pl.BlockSpec((1, DM), lambda i: (0, 0)),
            pl.BlockSpec((DM, DM), lambda i: (0, 0)),
            pl.BlockSpec((DM, 16), lambda i: (0, 0)),
        ],
        out_specs=[
            pl.BlockSpec((NHP, BN2, 128), lambda i: (0, i, 0)),
            pl.BlockSpec((BN2, 16), lambda i: (i, 0)),
        ],
        out_shape=[
            jax.ShapeDtypeStruct((NHP, NACC, 128), jnp.float32),
            jax.ShapeDtypeStruct((NACC, 16), jnp.float32),
        ],
    )(t, su, ss, g, be, W, ABm)


def _tc_out_body(t_ref, su_ref, ss_ref, g_ref, be_ref, w_ref, b_ref, o_ref):
    mean = su_ref[...] / N
    var = ss_ref[...] / N - mean * mean
    xh = g_ref[...] * (t_ref[...] - mean) * lax.rsqrt(var + 1e-5) + be_ref[...]
    o_ref[...] = jnp.dot(xh, w_ref[...],
                         preferred_element_type=jnp.float32) + b_ref[...]


def _tc_out(t, su, ss, g, be, Wp, bp):
    return pl.pallas_call(
        _tc_out_body,
        grid=(NB2,),
        in_specs=[
            pl.BlockSpec((BN2, DM), lambda i: (i, 0)),
            pl.BlockSpec((1, DM), lambda i: (0, 0)),
            pl.BlockSpec((1, DM), lambda i: (0, 0)),
            pl.BlockSpec((1, DM), lambda i: (0, 0)),
            pl.BlockSpec((1, DM), lambda i: (0, 0)),
            pl.BlockSpec((DM, 128), lambda i: (0, 0)),
            pl.BlockSpec((1, 128), lambda i: (0, 0)),
        ],
        out_specs=pl.BlockSpec((BN2, 128), lambda i: (i, 0)),
        out_shape=jax.ShapeDtypeStruct((NACC, 128), jnp.float32),
    )(t, su, ss, g, be, Wp, bp)


def _mixing_matrix(a_src, a_dst):
    idx = jnp.arange(HEADS)
    asr = jnp.zeros((HEADS, HID, HEADS), jnp.float32).at[idx, :, idx].set(a_src)
    ads = jnp.zeros((HEADS, HID, HEADS), jnp.float32).at[idx, :, idx].set(a_dst)
    asr = asr.reshape(DM, HEADS)
    ads = ads.reshape(DM, HEADS)
    cols = []
    for hp in range(NHP):
        cols += [asr[:, 2 * hp], asr[:, 2 * hp + 1],
                 ads[:, 2 * hp], ads[:, 2 * hp + 1]]
    return jnp.stack(cols, axis=1)


def _pack_alpha(ab):
    a = ab.reshape(ab.shape[0], NHP, 2, 2).astype(jnp.float16)
    packed = jax.lax.bitcast_convert_type(a, jnp.int32)
    return packed.transpose(1, 0, 2).reshape(NHP, ab.shape[0] * 2)


def kernel(x, edge_index, W0, a_src0, a_dst0, b0, g0, be0,
           W1, a_src1, a_dst1, b1, g1, be1, Wc, bc):
    eidx = (jnp.stack([edge_index[0].reshape(NT, NCH, C),
                       edge_index[1].reshape(NT, NCH, C)], axis=2)
            .reshape(2 * E))

    AB0 = _mixing_matrix(a_src0, a_dst0)
    AB1 = _mixing_matrix(a_src1, a_dst1)
    b0r = b0.reshape(1, DM)
    b1r = b1.reshape(1, DM)
    g0r, be0r = g0.reshape(1, DM), be0.reshape(1, DM)
    g1r, be1r = g1.reshape(1, DM), be1.reshape(1, DM)
    Wcp = jnp.zeros((DM, 128), jnp.float32).at[:, :Wc.shape[1]].set(Wc)
    bcp = jnp.zeros((1, 128), jnp.float32).at[0, :Wc.shape[1]].set(bc)

    zrows = jnp.zeros((128, 128), jnp.float32)

    h0p, ab0 = _tc_in(x, W0, AB0)
    num0 = _sc_edge_layer(h0p, _pack_alpha(ab0), eidx, zrows)
    den0 = num0[:, N:N + ND].reshape(NHP, ND * 64, 2)[:, :NACC]

    t0, su0, ss0 = _tc_norm(num0, den0, b0r)

    h1p, ab1 = _tc_mid(t0, su0, ss0, g0r, be0r, W1, AB1)
    num1 = _sc_edge_layer(h1p, _pack_alpha(ab1[:N]), eidx, zrows)
    den1 = num1[:, N:N + ND].reshape(NHP, ND * 64, 2)[:, :NACC]
    t1, su1, ss1 = _tc_norm(num1, den1, b1r)

    logits = _tc_out(t1, su1, ss1, g1r, be1r, Wcp, bcp)
    return logits[:N, :Wc.shape[1]]
